# Initial kernel scaffold; baseline (speedup 1.0000x reference)
import jax
import jax.numpy as jnp
from jax.experimental import pallas as pl


def kernel(x_y_z, points, params):
    raise NotImplementedError("write your pallas kernel here")


# speedup vs baseline: 22.8702x; 22.8702x over previous
import jax
import jax.numpy as jnp
from jax.experimental import pallas as pl

_RADII = (0.1, 0.2, 0.4)
_NSAMPLES = (16, 32, 64)
_NPOINT = 512
_S_BLK = 32
_BN_EPS = 1e-5


def _fps_body(xyz_ref, out_ref):
    B = xyz_ref.shape[0]
    N = xyz_ref.shape[2]
    x = xyz_ref[:, 0, :]
    y = xyz_ref[:, 1, :]
    z = xyz_ref[:, 2, :]
    lane = jax.lax.broadcasted_iota(jnp.int32, (B, N), 1)
    zeros1 = jnp.zeros((B, 1, 1), jnp.float32)

    def body(i, carry):
        distance, far = carry
        sel = lane == far
        cx = jnp.sum(jnp.where(sel, x, 0.0), axis=1, keepdims=True)
        cy = jnp.sum(jnp.where(sel, y, 0.0), axis=1, keepdims=True)
        cz = jnp.sum(jnp.where(sel, z, 0.0), axis=1, keepdims=True)
        row = jnp.concatenate(
            [zeros1, zeros1, zeros1,
             cx[:, :, None], cy[:, :, None], cz[:, :, None],
             zeros1, zeros1], axis=2)
        out_ref[:, pl.ds(i, 1), :] = row
        dx = x - cx
        dy = y - cy
        dz = z - cz
        d = (dx * dx + dy * dy) + dz * dz
        distance = jnp.minimum(distance, d)
        m = jnp.max(distance, axis=1, keepdims=True)
        far = jnp.min(jnp.where(distance == m, lane, N), axis=1, keepdims=True)
        return distance, far

    init = (jnp.full((B, N), 1e10, jnp.float32), jnp.zeros((B, 1), jnp.int32))
    jax.lax.fori_loop(0, _NPOINT, body, init)


def _group_mlp_body(newxyz_ref, xyzr_ref, vals_ref, *wrefs):
    out_ref = wrefs[-1]
    wlist = wrefs[:9]
    blist = wrefs[9:18]
    Sb = newxyz_ref.shape[1]
    N = xyzr_ref.shape[2]

    src = newxyz_ref[0]
    dstm = xyzr_ref[0]
    vals = vals_ref[0]

    sx = src[:, 3:4]
    sy = src[:, 4:5]
    sz = src[:, 5:6]
    src2 = (sx * sx + sy * sy) + sz * sz
    px = dstm[3:4, :]
    py = dstm[4:5, :]
    pz = dstm[5:6, :]
    dst2 = (px * px + py * py) + pz * pz
    dot = jnp.dot(src, dstm, preferred_element_type=jnp.float32)
    sqd = (src2 + dst2) - 2.0 * dot

    col_off = 0
    for bi, (r, K) in enumerate(zip(_RADII, _NSAMPLES)):
        valid = sqd <= r * r
        vi = valid.astype(jnp.int32)
        csum = vi
        d = 1
        while d < N:
            shifted = jnp.concatenate(
                [jnp.zeros((Sb, d), jnp.int32), csum[:, :N - d]], axis=1)
            csum = csum + shifted
            d *= 2
        rank = csum - vi
        count = csum[:, N - 1:N]
        slot = jnp.where(valid & (rank < K), rank, K)

        slot3 = jnp.broadcast_to(slot[:, None, :], (Sb, K, N))
        kio = jax.lax.broadcasted_iota(jnp.int32, (Sb, K, N), 1)
        sel = jnp.where(slot3 == kio, 1.0, 0.0).reshape(Sb * K, N)

        w1, w2, w3 = (wlist[3 * bi][...], wlist[3 * bi + 1][...],
                      wlist[3 * bi + 2][...])
        b1, b2, b3 = (blist[3 * bi][...], blist[3 * bi + 1][...],
                      blist[3 * bi + 2][...])
        o1 = w1.shape[1]
        vw1 = jnp.dot(vals, w1, preferred_element_type=jnp.float32)
        g = jnp.dot(sel, vw1, preferred_element_type=jnp.float32)
        c1 = b1 - jnp.dot(src, w1, preferred_element_type=jnp.float32)
        z = g.reshape(Sb, K, o1) + c1[:, None, :]
        kv = jax.lax.broadcasted_iota(jnp.int32, (Sb, K, o1), 1)
        z = jnp.where(kv < count[:, :, None], z, z[:, 0:1, :])
        h = jnp.maximum(z.reshape(Sb * K, o1), 0.0)
        h = jnp.maximum(jnp.dot(h, w2, preferred_element_type=jnp.float32) + b2, 0.0)
        h = jnp.maximum(jnp.dot(h, w3, preferred_element_type=jnp.float32) + b3, 0.0)
        o3 = w3.shape[1]
        mp = jnp.max(h.reshape(Sb, K, o3), axis=1)
        out_ref[0, :, col_off:col_off + o3] = mp
        col_off += o3


def kernel(x_y_z, points, params):
    B, _, N = x_y_z.shape
    S = _NPOINT

    wflat = []
    bflat = []
    for branch in params:
        for li, (W, b, gamma, beta, rm, rv) in enumerate(branch):
            sc = gamma / jnp.sqrt(rv + _BN_EPS)
            wt = (W * sc[:, None]).T.astype(jnp.float32)
            be = ((b - rm) * sc + beta).astype(jnp.float32)
            if li == 0:
                wt = jnp.concatenate(
                    [wt, jnp.zeros((2, wt.shape[1]), jnp.float32)], axis=0)
            wflat.append(wt)
            bflat.append(be[None, :])

    zeros3 = jnp.zeros((B, 3, N), jnp.float32)
    zeros2 = jnp.zeros((B, 2, N), jnp.float32)
    xyzr8 = jnp.concatenate([zeros3, x_y_z, zeros2], axis=1)
    vals8 = jnp.transpose(
        jnp.concatenate([points, x_y_z, zeros2], axis=1), (0, 2, 1))

    newxyz8 = pl.pallas_call(
        _fps_body,
        out_shape=jax.ShapeDtypeStruct((B, S, 8), jnp.float32),
    )(x_y_z)

    nsb = S // _S_BLK
    wspecs = [pl.BlockSpec(w.shape, lambda b, s: (0, 0)) for w in wflat]
    bspecs = [pl.BlockSpec(bb.shape, lambda b, s: (0, 0)) for bb in bflat]
    out_pts = pl.pallas_call(
        _group_mlp_body,
        grid=(B, nsb),
        in_specs=[
            pl.BlockSpec((1, _S_BLK, 8), lambda b, s: (b, s, 0)),
            pl.BlockSpec((1, 8, N), lambda b, s: (b, 0, 0)),
            pl.BlockSpec((1, N, 8), lambda b, s: (b, 0, 0)),
        ] + wspecs + bspecs,
        out_specs=pl.BlockSpec((1, _S_BLK, 320), lambda b, s: (b, s, 0)),
        out_shape=jax.ShapeDtypeStruct((B, S, 320), jnp.float32),
    )(newxyz8, xyzr8, vals8, *wflat, *bflat)

    new_xyz_out = jnp.transpose(newxyz8[:, :, 3:6], (0, 2, 1))
    new_points = jnp.transpose(out_pts, (0, 2, 1))
    return (new_xyz_out, new_points)
